# Initial kernel scaffold; baseline (speedup 1.0000x reference)
import jax
import jax.numpy as jnp
from jax.experimental import pallas as pl


def kernel(queries, keys, values):
    raise NotImplementedError("write your pallas kernel here")


# speedup vs baseline: 3.8142x; 3.8142x over previous
import functools
import math

import jax
import jax.numpy as jnp
import numpy as np
from jax.experimental import pallas as pl
from jax.experimental.pallas import tpu as pltpu

_B, _L, _H, _D = 4, 2048, 16, 64
_FACTOR = 5
_U = min(_FACTOR * int(math.ceil(math.log(_L))), _L)

_HIGH = jax.lax.Precision.HIGHEST


def _count_matrix() -> np.ndarray:
    idx = np.asarray(
        jax.random.randint(jax.random.key(42), (_L, _U), 0, _L)
    )
    c = np.zeros((_L, _L), np.int8)
    np.add.at(c, (np.arange(_L)[:, None], idx), 1)
    return c


_COUNTS = _count_matrix()


def _m_kernel(c_ref, q_ref, k_ref, m_ref, *, tl):
    lt = pl.program_id(2)
    q = q_ref[0, 0, :, :].astype(jnp.bfloat16)
    k = k_ref[0, 0, :, :].astype(jnp.bfloat16)
    s = jax.lax.dot_general(
        q, k, (((1,), (1,)), ((), ())),
        preferred_element_type=jnp.float32)
    cnt = c_ref[pl.ds(lt * tl, tl), :].astype(jnp.float32)
    masked = jnp.where(cnt > 0.0, s, -1e30)
    m_max = jnp.max(masked, axis=1)
    m_sum = jnp.sum(s * cnt, axis=1)
    m_ref[0, 0, :] = m_max - m_sum * (1.0 / _L)


def _topk_kernel(m_ref, out_ref):
    rows = _B * _H
    m = m_ref[...]
    iota_l = jax.lax.broadcasted_iota(jnp.int32, (rows, _L), 1)
    iota_u = jax.lax.broadcasted_iota(jnp.int32, (rows, 128), 1)

    def body(i, carry):
        cur, acc = carry
        vmax = jnp.max(cur, axis=1, keepdims=True)
        cand = jnp.where(cur == vmax, iota_l, _L)
        amin = jnp.min(cand, axis=1, keepdims=True)
        acc = jnp.where(iota_u == i, amin, acc)
        cur = jnp.where(iota_l == amin, -jnp.inf, cur)
        return cur, acc

    _, acc = jax.lax.fori_loop(
        0, _U, body, (m, jnp.zeros((rows, 128), jnp.int32)))
    out_ref[...] = acc


def _attn_kernel(mtop_ref, q_ref, k_ref, v_ref, out_ref, qr_ref):
    b = pl.program_id(0)
    h = pl.program_id(1)
    bh = b * _H + h
    k = k_ref[0, 0, :, :]
    v = v_ref[0, 0, :, :]

    for i in range(_U):
        qr_ref[i, :] = q_ref[0, 0, mtop_ref[bh, i], :]
    qr = qr_ref[...]

    s = jax.lax.dot_general(
        qr.astype(jnp.bfloat16), k.astype(jnp.bfloat16),
        (((1,), (1,)), ((), ())),
        preferred_element_type=jnp.float32)
    s = s * (1.0 / math.sqrt(_D))
    smax = jnp.max(s, axis=1, keepdims=True)
    e = jnp.exp(s - smax)
    p = e / jnp.sum(e, axis=1, keepdims=True)
    upd = jax.lax.dot_general(
        p.astype(jnp.bfloat16), v.astype(jnp.bfloat16),
        (((1,), (0,)), ((), ())),
        preferred_element_type=jnp.float32)

    vmean = jnp.mean(v, axis=0, keepdims=True)
    out_ref[0, 0, :, :] = jnp.broadcast_to(vmean, (_L, _D))
    for i in range(_U):
        out_ref[0, 0, pl.ds(mtop_ref[bh, i], 1), :] = upd[i:i + 1, :]


def kernel(queries, keys, values):
    B, L, H, D = queries.shape
    cnt = jnp.asarray(_COUNTS)
    qt = jnp.transpose(queries, (0, 2, 1, 3))
    kt = jnp.transpose(keys, (0, 2, 1, 3))
    vt = jnp.transpose(values, (0, 2, 1, 3))

    tl = 128
    m = pl.pallas_call(
        functools.partial(_m_kernel, tl=tl),
        grid=(B, H, L // tl),
        in_specs=[
            pl.BlockSpec((L, L), lambda b, h, lt: (0, 0)),
            pl.BlockSpec((1, 1, tl, D), lambda b, h, lt: (b, h, lt, 0)),
            pl.BlockSpec((1, 1, L, D), lambda b, h, lt: (b, h, 0, 0)),
        ],
        out_specs=pl.BlockSpec((1, 1, tl), lambda b, h, lt: (b * H + h, 0, lt)),
        out_shape=jax.ShapeDtypeStruct((B * H, 1, L), jnp.float32),
    )(cnt, qt, kt)

    m2 = m.reshape(B * H, L)
    mtop = pl.pallas_call(
        _topk_kernel,
        in_specs=[pl.BlockSpec((B * H, L), lambda: (0, 0))],
        out_specs=pl.BlockSpec((B * H, 128), lambda: (0, 0)),
        out_shape=jax.ShapeDtypeStruct((B * H, 128), jnp.int32),
    )(m2)

    out = pl.pallas_call(
        _attn_kernel,
        grid_spec=pltpu.PrefetchScalarGridSpec(
            num_scalar_prefetch=1,
            grid=(B, H),
            in_specs=[
                pl.BlockSpec((1, 1, L, D), lambda b, h, mt: (b, h, 0, 0)),
                pl.BlockSpec((1, 1, L, D), lambda b, h, mt: (b, h, 0, 0)),
                pl.BlockSpec((1, 1, L, D), lambda b, h, mt: (b, h, 0, 0)),
            ],
            out_specs=pl.BlockSpec((1, 1, L, D), lambda b, h, mt: (b, h, 0, 0)),
            scratch_shapes=[pltpu.VMEM((_U, D), jnp.float32)],
        ),
        out_shape=jax.ShapeDtypeStruct((B, H, L, D), jnp.float32),
    )(mtop, qt, kt, vt)
    return jnp.transpose(out, (0, 2, 1, 3))


# speedup vs baseline: 6.1173x; 1.6038x over previous
import functools
import math

import jax
import jax.numpy as jnp
import numpy as np
from jax.experimental import pallas as pl
from jax.experimental.pallas import tpu as pltpu

_B, _L, _H, _D = 4, 2048, 16, 64
_FACTOR = 5
_U = min(_FACTOR * int(math.ceil(math.log(_L))), _L)

_HIGH = jax.lax.Precision.HIGHEST


def _count_matrix() -> np.ndarray:
    idx = np.asarray(
        jax.random.randint(jax.random.key(42), (_L, _U), 0, _L)
    )
    c = np.zeros((_L, _L), np.int8)
    np.add.at(c, (np.arange(_L)[:, None], idx), 1)
    return c


_COUNTS = _count_matrix()
_COUNTS_F32 = _COUNTS.astype(np.float32)
_NEGMASK = np.where(_COUNTS > 0, 0.0, -1e30).astype(np.float32)


def _m_kernel(c_ref, nm_ref, q_ref, k_ref, m_ref, *, tl):
    lt = pl.program_id(2)
    q = q_ref[0, 0, :, :].astype(jnp.bfloat16)
    k = k_ref[0, 0, :, :].astype(jnp.bfloat16)
    s = jax.lax.dot_general(
        q, k, (((1,), (1,)), ((), ())),
        preferred_element_type=jnp.float32)
    cnt = c_ref[pl.ds(lt * tl, tl), :]
    neg = nm_ref[pl.ds(lt * tl, tl), :]
    m_max = jnp.max(s + neg, axis=1)
    m_sum = jnp.sum(s * cnt, axis=1)
    m_ref[0, 0, :] = m_max - m_sum * (1.0 / _L)


def _topk_kernel(m_ref, out_ref):
    rows = _B * _H
    m = m_ref[...]
    iota_l = jax.lax.broadcasted_iota(jnp.int32, (rows, _L), 1)
    iota_u = jax.lax.broadcasted_iota(jnp.int32, (rows, 128), 1)

    def body(i, carry):
        cur, acc = carry
        vmax = jnp.max(cur, axis=1, keepdims=True)
        cand = jnp.where(cur == vmax, iota_l, _L)
        amin = jnp.min(cand, axis=1, keepdims=True)
        acc = jnp.where(iota_u == i, amin, acc)
        cur = jnp.where(iota_l == amin, -jnp.inf, cur)
        return cur, acc

    _, acc = jax.lax.fori_loop(
        0, _U, body, (m, jnp.zeros((rows, 128), jnp.int32)))
    out_ref[...] = acc


def _attn_kernel(mtop_ref, q_ref, k_ref, v_ref, out_ref, qr_ref):
    b = pl.program_id(0)
    h = pl.program_id(1)
    bh = b * _H + h
    k = k_ref[0, 0, :, :]
    v = v_ref[0, 0, :, :]

    for i in range(_U):
        qr_ref[i, :] = q_ref[0, 0, mtop_ref[bh, i], :]
    qr = qr_ref[...]

    s = jax.lax.dot_general(
        qr.astype(jnp.bfloat16), k.astype(jnp.bfloat16),
        (((1,), (1,)), ((), ())),
        preferred_element_type=jnp.float32)
    s = s * (1.0 / math.sqrt(_D))
    smax = jnp.max(s, axis=1, keepdims=True)
    e = jnp.exp(s - smax)
    p = e / jnp.sum(e, axis=1, keepdims=True)
    upd = jax.lax.dot_general(
        p.astype(jnp.bfloat16), v.astype(jnp.bfloat16),
        (((1,), (0,)), ((), ())),
        preferred_element_type=jnp.float32)

    vmean = jnp.mean(v, axis=0, keepdims=True)
    out_ref[0, 0, :, :] = jnp.broadcast_to(vmean, (_L, _D))
    for i in range(_U):
        out_ref[0, 0, pl.ds(mtop_ref[bh, i], 1), :] = upd[i:i + 1, :]


def kernel(queries, keys, values):
    B, L, H, D = queries.shape
    cnt = jnp.asarray(_COUNTS_F32)
    neg = jnp.asarray(_NEGMASK)
    qt = jnp.transpose(queries, (0, 2, 1, 3))
    kt = jnp.transpose(keys, (0, 2, 1, 3))
    vt = jnp.transpose(values, (0, 2, 1, 3))

    tl = 512
    m = pl.pallas_call(
        functools.partial(_m_kernel, tl=tl),
        grid=(B, H, L // tl),
        in_specs=[
            pl.BlockSpec((L, L), lambda b, h, lt: (0, 0)),
            pl.BlockSpec((L, L), lambda b, h, lt: (0, 0)),
            pl.BlockSpec((1, 1, tl, D), lambda b, h, lt: (b, h, lt, 0)),
            pl.BlockSpec((1, 1, L, D), lambda b, h, lt: (b, h, 0, 0)),
        ],
        out_specs=pl.BlockSpec((1, 1, tl), lambda b, h, lt: (b * H + h, 0, lt)),
        out_shape=jax.ShapeDtypeStruct((B * H, 1, L), jnp.float32),
    )(cnt, neg, qt, kt)

    m2 = m.reshape(B * H, L)
    mtop = pl.pallas_call(
        _topk_kernel,
        in_specs=[pl.BlockSpec((B * H, L), lambda: (0, 0))],
        out_specs=pl.BlockSpec((B * H, 128), lambda: (0, 0)),
        out_shape=jax.ShapeDtypeStruct((B * H, 128), jnp.int32),
    )(m2)

    out = pl.pallas_call(
        _attn_kernel,
        grid_spec=pltpu.PrefetchScalarGridSpec(
            num_scalar_prefetch=1,
            grid=(B, H),
            in_specs=[
                pl.BlockSpec((1, 1, L, D), lambda b, h, mt: (b, h, 0, 0)),
                pl.BlockSpec((1, 1, L, D), lambda b, h, mt: (b, h, 0, 0)),
                pl.BlockSpec((1, 1, L, D), lambda b, h, mt: (b, h, 0, 0)),
            ],
            out_specs=pl.BlockSpec((1, 1, L, D), lambda b, h, mt: (b, h, 0, 0)),
            scratch_shapes=[pltpu.VMEM((_U, D), jnp.float32)],
        ),
        out_shape=jax.ShapeDtypeStruct((B, H, L, D), jnp.float32),
    )(mtop, qt, kt, vt)
    return jnp.transpose(out, (0, 2, 1, 3))
